# baseline (speedup 1.0000x reference)
import functools

import jax
import jax.numpy as jnp
from jax import lax
from jax.experimental import pallas as pl
from jax.experimental.pallas import tpu as pltpu
from jax.experimental.pallas import tpu_sc as plsc


def _argmax_body(vocab, x_ref, out_ref):
    blk = x_ref[...]
    m = jnp.max(blk, axis=-1, keepdims=True)
    pos = lax.broadcasted_iota(jnp.int32, blk.shape, 1)
    idx = jnp.min(jnp.where(blk == m, pos, vocab), axis=-1)
    out_ref[...] = idx.reshape(1, 1, -1)


def _tc_argmax(x2, bm):
    m, v = x2.shape
    nb = m // bm
    out = pl.pallas_call(
        functools.partial(_argmax_body, v),
        grid=(nb,),
        in_specs=[pl.BlockSpec((bm, v), lambda i: (i, 0))],
        out_specs=pl.BlockSpec((1, 1, bm), lambda i: (i, 0, 0)),
        out_shape=jax.ShapeDtypeStruct((nb, 1, bm), jnp.int32),
    )(x2)
    return out.reshape(m)


def _sc_gather(table, tokens, n_chunks=2):
    m = tokens.shape[0]
    d = table.shape[1]
    info = plsc.get_sparse_core_info()
    nc, ns = info.num_cores, info.num_subcores
    nw = nc * ns
    b_per_w = m // nw
    bc = b_per_w // n_chunks
    mesh = plsc.VectorSubcoreMesh(core_axis_name="c", subcore_axis_name="s")

    @functools.partial(
        pl.kernel,
        mesh=mesh,
        out_type=jax.ShapeDtypeStruct((m, d), jnp.float32),
        scratch_types=[
            pltpu.VMEM((b_per_w,), jnp.int32),
            pltpu.VMEM((bc, d), jnp.float32),
            pltpu.SemaphoreType.DMA,
        ],
    )
    def _gather(table_hbm, idx_hbm, out_hbm, idx_v, rows_v, sem):
        wid = lax.axis_index("s") * nc + lax.axis_index("c")
        base = wid * b_per_w
        pltpu.sync_copy(idx_hbm.at[pl.ds(base, b_per_w)], idx_v)
        for c in range(n_chunks):
            pltpu.async_copy(
                table_hbm.at[idx_v.at[pl.ds(c * bc, bc)]], rows_v, sem
            ).wait()
            pltpu.sync_copy(rows_v, out_hbm.at[pl.ds(base + c * bc, bc)])

    return _gather(table, tokens)


def kernel(x, W):
    b, n, v = x.shape
    d = W.shape[1]
    x2 = x.reshape(b * n, v)
    tokens = _tc_argmax(x2, bm=1024)
    w128 = jnp.pad(W, ((0, 0), (0, 128 - d)))
    out = _sc_gather(w128, tokens)
    return out[:, :d].reshape(b, n, d)


# speedup vs baseline: 1.2281x; 1.2281x over previous
import functools

import jax
import jax.numpy as jnp
from jax import lax
from jax.experimental import pallas as pl
from jax.experimental.pallas import tpu as pltpu
from jax.experimental.pallas import tpu_sc as plsc


def _argmax_body(vocab, x_ref, out_ref):
    blk = x_ref[...]
    m = jnp.max(blk, axis=-1, keepdims=True)
    pos = lax.broadcasted_iota(jnp.int32, blk.shape, 2)
    idx = jnp.min(jnp.where(blk == m, pos, vocab), axis=-1)
    out_ref[...] = idx


def _tc_argmax(x, bb):
    b, n, v = x.shape
    nb = b // bb
    return pl.pallas_call(
        functools.partial(_argmax_body, v),
        grid=(nb,),
        in_specs=[pl.BlockSpec((bb, n, v), lambda i: (i, 0, 0))],
        out_specs=pl.BlockSpec((bb, n), lambda i: (i, 0)),
        out_shape=jax.ShapeDtypeStruct((b, n), jnp.int32),
    )(x)


def _sc_gather(table, tokens, n_chunks=2):
    m = tokens.shape[0]
    d = table.shape[1]
    info = plsc.get_sparse_core_info()
    nc, ns = info.num_cores, info.num_subcores
    nw = nc * ns
    b_per_w = m // nw
    bc = b_per_w // n_chunks
    mesh = plsc.VectorSubcoreMesh(core_axis_name="c", subcore_axis_name="s")

    @functools.partial(
        pl.kernel,
        mesh=mesh,
        out_type=jax.ShapeDtypeStruct((m, d), jnp.float32),
        scratch_types=[
            pltpu.VMEM((b_per_w,), jnp.int32),
            pltpu.VMEM((bc, d), jnp.float32),
            pltpu.SemaphoreType.DMA,
        ],
    )
    def _gather(table_hbm, idx_hbm, out_hbm, idx_v, rows_v, sem):
        wid = lax.axis_index("s") * nc + lax.axis_index("c")
        base = wid * b_per_w
        pltpu.sync_copy(idx_hbm.at[pl.ds(base, b_per_w)], idx_v)
        for c in range(n_chunks):
            pltpu.async_copy(
                table_hbm.at[idx_v.at[pl.ds(c * bc, bc)]], rows_v, sem
            ).wait()
            pltpu.sync_copy(rows_v, out_hbm.at[pl.ds(base + c * bc, bc)])

    return _gather(table, tokens)


def kernel(x, W):
    b, n, v = x.shape
    d = W.shape[1]
    tokens = _tc_argmax(x, bb=16).reshape(b * n)
    w128 = jnp.pad(W, ((0, 0), (0, 128 - d)))
    out = _sc_gather(w128, tokens)
    return out[:, :d].reshape(b, n, d)


# speedup vs baseline: 1.2902x; 1.0505x over previous
import functools

import jax
import jax.numpy as jnp
from jax import lax
from jax.experimental import pallas as pl
from jax.experimental.pallas import tpu as pltpu
from jax.experimental.pallas import tpu_sc as plsc


def _argmax_body(vocab, x_ref, out_ref):
    blk = x_ref[...]
    m = jnp.max(blk, axis=-1, keepdims=True)
    pos = lax.broadcasted_iota(jnp.int32, blk.shape, 2)
    idx = jnp.min(jnp.where(blk == m, pos, vocab), axis=-1)
    out_ref[...] = idx


def _tc_argmax(x, bb):
    b, n, v = x.shape
    nb = b // bb
    return pl.pallas_call(
        functools.partial(_argmax_body, v),
        grid=(nb,),
        in_specs=[pl.BlockSpec((bb, n, v), lambda i: (i, 0, 0))],
        out_specs=pl.BlockSpec((bb, n), lambda i: (i, 0)),
        out_shape=jax.ShapeDtypeStruct((b, n), jnp.int32),
    )(x)


def _sc_gather(table, tokens, n_chunks=1, untiled=False):
    m = tokens.shape[0]
    d = table.shape[1]
    info = plsc.get_sparse_core_info()
    nc, ns = info.num_cores, info.num_subcores
    nw = nc * ns
    b_per_w = m // nw
    bc = b_per_w // n_chunks
    mesh = plsc.VectorSubcoreMesh(core_axis_name="c", subcore_axis_name="s")
    params = pltpu.CompilerParams(use_tc_tiling_on_sc=False) if untiled else None

    @functools.partial(
        pl.kernel,
        mesh=mesh,
        out_type=jax.ShapeDtypeStruct((m, d), jnp.float32),
        scratch_types=[
            pltpu.VMEM((b_per_w,), jnp.int32),
            pltpu.VMEM((bc, d), jnp.float32),
            pltpu.SemaphoreType.DMA,
        ],
        compiler_params=params,
    )
    def _gather(table_hbm, idx_hbm, out_hbm, idx_v, rows_v, sem):
        wid = lax.axis_index("s") * nc + lax.axis_index("c")
        base = wid * b_per_w
        pltpu.sync_copy(idx_hbm.at[pl.ds(base, b_per_w)], idx_v)
        for c in range(n_chunks):
            src = idx_v if n_chunks == 1 else idx_v.at[pl.ds(c * bc, bc)]
            pltpu.async_copy(table_hbm.at[src], rows_v, sem).wait()
            pltpu.sync_copy(rows_v, out_hbm.at[pl.ds(base + c * bc, bc)])

    return _gather(table, tokens)


def kernel(x, W):
    b, n, v = x.shape
    d = W.shape[1]
    tokens = _tc_argmax(x, bb=16).reshape(b * n)
    out = _sc_gather(W, tokens, n_chunks=1, untiled=True)
    return out.reshape(b, n, d)
